# baseline (speedup 1.0000x reference)
import functools

import jax
import jax.numpy as jnp
from jax.experimental import pallas as pl
from jax.experimental.pallas import tpu as pltpu

B = 128
V = 100000
V_BLK = 4096
GRID = (V + V_BLK - 1) // V_BLK

_NEG_INF = float("-inf")


def _fused_kernel(logits_ref, gumbel_ref, out_ref, logp_ref, ent_ref,
                  m_ref, s_ref, t_ref, bx_ref, bl_ref, bi_ref):
    i = pl.program_id(0)

    @pl.when(i == 0)
    def _init():
        m_ref[...] = jnp.full((B, 1), _NEG_INF, jnp.float32)
        s_ref[...] = jnp.zeros((B, 1), jnp.float32)
        t_ref[...] = jnp.zeros((B, 1), jnp.float32)
        bx_ref[...] = jnp.full((B, 1), _NEG_INF, jnp.float32)
        bl_ref[...] = jnp.zeros((B, 1), jnp.float32)
        bi_ref[...] = jnp.zeros((B, 1), jnp.int32)

    l = logits_ref[...]
    u = gumbel_ref[...]

    col = jax.lax.broadcasted_iota(jnp.int32, (B, V_BLK), 1)
    valid = (i * V_BLK + col) < V
    l = jnp.where(valid, l, _NEG_INF)

    blk_max = jnp.max(l, axis=1, keepdims=True)
    m_old = m_ref[...]
    m_new = jnp.maximum(m_old, blk_max)
    e = jnp.exp(l - m_new)
    scale = jnp.exp(m_old - m_new)
    s_ref[...] = s_ref[...] * scale + jnp.sum(e, axis=1, keepdims=True)
    le = jnp.where(valid, l * e, 0.0)
    t_ref[...] = t_ref[...] * scale + jnp.sum(le, axis=1, keepdims=True)
    m_ref[...] = m_new

    g = -jnp.log(-jnp.log(u))
    x = jnp.where(valid, l + g, _NEG_INF)
    blk_bx = jnp.max(x, axis=1, keepdims=True)
    at_max = x == blk_bx
    blk_bi = jnp.min(jnp.where(at_max, col, V), axis=1, keepdims=True)
    first = col == blk_bi
    blk_bl = jnp.sum(jnp.where(first, l, 0.0), axis=1, keepdims=True)
    better = blk_bx > bx_ref[...]
    bx_ref[...] = jnp.where(better, blk_bx, bx_ref[...])
    bl_ref[...] = jnp.where(better, blk_bl, bl_ref[...])
    bi_ref[...] = jnp.where(better, blk_bi + i * V_BLK, bi_ref[...])

    @pl.when(i == GRID - 1)
    def _finish():
        lse = m_ref[...] + jnp.log(s_ref[...])
        out_ref[...] = bi_ref[...]
        logp_ref[...] = bl_ref[...] - lse
        ent_ref[...] = lse - t_ref[...] / s_ref[...]


@functools.partial(jax.jit, static_argnames=())
def kernel(logits, gumbel_u):
    out2, logp2, ent2 = pl.pallas_call(
        _fused_kernel,
        grid=(GRID,),
        in_specs=[
            pl.BlockSpec((B, V_BLK), lambda i: (0, i)),
            pl.BlockSpec((B, V_BLK), lambda i: (0, i)),
        ],
        out_specs=[
            pl.BlockSpec((B, 1), lambda i: (0, 0)),
            pl.BlockSpec((B, 1), lambda i: (0, 0)),
            pl.BlockSpec((B, 1), lambda i: (0, 0)),
        ],
        out_shape=[
            jax.ShapeDtypeStruct((B, 1), jnp.int32),
            jax.ShapeDtypeStruct((B, 1), jnp.float32),
            jax.ShapeDtypeStruct((B, 1), jnp.float32),
        ],
        scratch_shapes=[
            pltpu.VMEM((B, 1), jnp.float32),
            pltpu.VMEM((B, 1), jnp.float32),
            pltpu.VMEM((B, 1), jnp.float32),
            pltpu.VMEM((B, 1), jnp.float32),
            pltpu.VMEM((B, 1), jnp.float32),
            pltpu.VMEM((B, 1), jnp.int32),
        ],
    )(logits, gumbel_u)
    return (out2[:, 0], logp2[:, 0], ent2[:, 0])
